# Initial kernel scaffold; baseline (speedup 1.0000x reference)
import jax
import jax.numpy as jnp
from jax import lax
from jax.experimental import pallas as pl
from jax.experimental.pallas import tpu as pltpu
from jax.experimental.pallas import tpu_sc as plsc

NC, NS, LANES = 2, 16, 16
NW = NC * NS
CHUNK = 100


def _sc_body(idx_hbm, tok_hbm, pos_hbm, out_hbm, idx_v, pos_v, buf_v, gsem):
    wid = lax.axis_index("s") * NC + lax.axis_index("c")
    n_chunks = idx_hbm.shape[0] // NW
    d_model = tok_hbm.shape[1]
    n_vecs = d_model // LANES

    pltpu.sync_copy(idx_hbm.at[pl.ds(wid * n_chunks, n_chunks)], idx_v)
    pltpu.sync_copy(pos_hbm, pos_v)

    def chunk_body(c, carry):
        g = wid * n_chunks + c
        pltpu.async_copy(tok_hbm.at[idx_v.at[c]], buf_v, gsem).wait()
        pos_base = lax.rem(c, 2) * CHUNK

        def row_body(r, rc):
            pr = pos_base + r
            for d in range(n_vecs):
                x = pos_v[pr, pl.ds(d * LANES, LANES)]
                plsc.addupdate(buf_v.at[r, pl.ds(d * LANES, LANES)], x)
            return rc

        lax.fori_loop(0, CHUNK, row_body, 0)
        pltpu.sync_copy(buf_v, out_hbm.at[pl.ds(g * CHUNK, CHUNK)])
        return carry

    lax.fori_loop(0, n_chunks, chunk_body, 0)


def kernel(inputs, token_table, pos_table):
    batch, seq_len = inputs.shape
    d_model = token_table.shape[1]
    total = batch * seq_len
    idx2d = inputs.reshape(total // CHUNK, CHUNK).astype(jnp.int32)
    n_chunks = (total // CHUNK) // NW

    mesh = plsc.VectorSubcoreMesh(core_axis_name="c", subcore_axis_name="s")
    out = pl.kernel(
        _sc_body,
        out_type=jax.ShapeDtypeStruct((total, d_model), jnp.float32),
        mesh=mesh,
        scratch_types=[
            pltpu.VMEM((n_chunks, CHUNK), jnp.int32),
            pltpu.VMEM((seq_len, d_model), jnp.float32),
            pltpu.VMEM((CHUNK, d_model), jnp.float32),
            pltpu.SemaphoreType.DMA,
        ],
    )(idx2d, token_table, pos_table)
    return out.reshape(batch, seq_len, d_model)


# speedup vs baseline: 1.7513x; 1.7513x over previous
import jax
import jax.numpy as jnp
from jax import lax
from jax.experimental import pallas as pl
from jax.experimental.pallas import tpu as pltpu
from jax.experimental.pallas import tpu_sc as plsc

NC, NS, LANES = 2, 16, 16
NW = NC * NS
CHUNK = 40
PERIOD = 200 // CHUNK


def _sc_body(idx_hbm, tok_hbm, pos_hbm, out_hbm, idx_v, pos_v, buf_v, gsem):
    wid = lax.axis_index("s") * NC + lax.axis_index("c")
    n_chunks = idx_hbm.shape[0] // NW
    d_model = tok_hbm.shape[1]
    n_vecs = d_model // LANES

    pltpu.sync_copy(idx_hbm.at[pl.ds(wid * n_chunks, n_chunks)], idx_v)
    pltpu.sync_copy(pos_hbm, pos_v)

    def chunk_body(c, carry):
        g = wid * n_chunks + c
        pltpu.async_copy(tok_hbm.at[idx_v.at[c]], buf_v, gsem).wait()
        pos_base = lax.rem(c, PERIOD) * CHUNK

        def row_body(r, rc):
            pr = pos_base + r
            for d in range(n_vecs):
                x = pos_v[pr, pl.ds(d * LANES, LANES)]
                plsc.addupdate(buf_v.at[r, pl.ds(d * LANES, LANES)], x)
            return rc

        lax.fori_loop(0, CHUNK, row_body, 0)
        pltpu.sync_copy(buf_v, out_hbm.at[pl.ds(g * CHUNK, CHUNK)])
        return carry

    lax.fori_loop(0, n_chunks, chunk_body, 0)


def kernel(inputs, token_table, pos_table):
    batch, seq_len = inputs.shape
    d_model = token_table.shape[1]
    total = batch * seq_len
    idx2d = inputs.reshape(total // CHUNK, CHUNK).astype(jnp.int32)
    n_chunks = (total // CHUNK) // NW

    mesh = plsc.VectorSubcoreMesh(core_axis_name="c", subcore_axis_name="s")
    out = pl.kernel(
        _sc_body,
        out_type=jax.ShapeDtypeStruct((total, d_model), jnp.float32),
        mesh=mesh,
        scratch_types=[
            pltpu.VMEM((n_chunks, CHUNK), jnp.int32),
            pltpu.VMEM((seq_len, d_model), jnp.float32),
            pltpu.VMEM((CHUNK, d_model), jnp.float32),
            pltpu.SemaphoreType.DMA,
        ],
    )(idx2d, token_table, pos_table)
    return out.reshape(batch, seq_len, d_model)


# speedup vs baseline: 3.1820x; 1.8169x over previous
import jax
import jax.numpy as jnp
from jax import lax
from jax.experimental import pallas as pl
from jax.experimental.pallas import tpu as pltpu
from jax.experimental.pallas import tpu_sc as plsc

NC, NS, LANES = 2, 16, 16
NW = NC * NS
CHUNK = 40
PERIOD = 200 // CHUNK
NBUF = 4
LOOK = 2


def _sc_body(idx_hbm, tok_hbm, pos_hbm, out_hbm, idx_v, pos_v, bufs, gsem, ssem):
    wid = lax.axis_index("s") * NC + lax.axis_index("c")
    n_chunks = idx_hbm.shape[0] // NW
    d_model = tok_hbm.shape[1]
    n_vecs = d_model // LANES
    base = wid * n_chunks

    pltpu.sync_copy(idx_hbm.at[pl.ds(base, n_chunks)], idx_v)
    pltpu.sync_copy(pos_hbm, pos_v)

    def start_gather(c):
        b = lax.rem(c, NBUF)
        pltpu.async_copy(tok_hbm.at[idx_v.at[c]], bufs.at[b], gsem.at[b])

    def wait_gather(c):
        b = lax.rem(c, NBUF)
        pltpu.make_async_copy(tok_hbm.at[idx_v.at[c]], bufs.at[b], gsem.at[b]).wait()

    def start_scatter(c):
        b = lax.rem(c, NBUF)
        pltpu.async_copy(
            bufs.at[b], out_hbm.at[pl.ds((base + c) * CHUNK, CHUNK)], ssem.at[b])

    def wait_scatter(c):
        b = lax.rem(c, NBUF)
        pltpu.make_async_copy(
            bufs.at[b], out_hbm.at[pl.ds((base + c) * CHUNK, CHUNK)], ssem.at[b]).wait()

    def compute(c):
        b = lax.rem(c, NBUF)
        pos_base = lax.rem(c, PERIOD) * CHUNK

        def row_body(r, rc):
            pr = pos_base + r
            for d in range(n_vecs):
                x = pos_v[pr, pl.ds(d * LANES, LANES)]
                plsc.addupdate(bufs.at[b, r, pl.ds(d * LANES, LANES)], x)
            return rc

        lax.fori_loop(0, CHUNK, row_body, 0)

    for i in range(LOOK):
        start_gather(i)
    for i in range(LOOK):
        start_gather(i + LOOK)
        wait_gather(i)
        compute(i)
        start_scatter(i)

    def steady(i, carry):
        wait_scatter(i - LOOK)
        start_gather(i + LOOK)
        wait_gather(i)
        compute(i)
        start_scatter(i)
        return carry

    lax.fori_loop(LOOK, n_chunks - LOOK, steady, 0)

    for i in range(n_chunks - LOOK, n_chunks):
        wait_scatter(i - LOOK)
        wait_gather(i)
        compute(i)
        start_scatter(i)
    for i in range(n_chunks - LOOK, n_chunks):
        wait_scatter(i)


def kernel(inputs, token_table, pos_table):
    batch, seq_len = inputs.shape
    d_model = token_table.shape[1]
    total = batch * seq_len
    idx2d = inputs.reshape(total // CHUNK, CHUNK).astype(jnp.int32)
    n_chunks = (total // CHUNK) // NW

    mesh = plsc.VectorSubcoreMesh(core_axis_name="c", subcore_axis_name="s")
    out = pl.kernel(
        _sc_body,
        out_type=jax.ShapeDtypeStruct((total, d_model), jnp.float32),
        mesh=mesh,
        scratch_types=[
            pltpu.VMEM((n_chunks, CHUNK), jnp.int32),
            pltpu.VMEM((seq_len, d_model), jnp.float32),
            pltpu.VMEM((NBUF, CHUNK, d_model), jnp.float32),
            pltpu.SemaphoreType.DMA((NBUF,)),
            pltpu.SemaphoreType.DMA((NBUF,)),
        ],
    )(idx2d, token_table, pos_table)
    return out.reshape(batch, seq_len, d_model)


# speedup vs baseline: 6.4394x; 2.0237x over previous
import jax
import jax.numpy as jnp
from jax import lax
from jax.experimental import pallas as pl
from jax.experimental.pallas import tpu as pltpu
from jax.experimental.pallas import tpu_sc as plsc

NC, NS, LANES = 2, 16, 16
NW = NC * NS
CHUNK = 40
PERIOD = 200 // CHUNK
NBUF = 4
LOOK = 2


def _sc_body(idx_hbm, tok_hbm, pos_hbm, out_hbm, idx_v, pos_v, bufs, gsem, ssem):
    wid = lax.axis_index("s") * NC + lax.axis_index("c")
    n_chunks = idx_hbm.shape[0] // NW
    d_model = tok_hbm.shape[1]
    n_vecs = d_model // LANES
    base = wid * n_chunks

    pltpu.sync_copy(idx_hbm.at[pl.ds(base, n_chunks)], idx_v)
    pltpu.sync_copy(pos_hbm, pos_v)

    def start_gather(c):
        b = lax.rem(c, NBUF)
        pltpu.async_copy(tok_hbm.at[idx_v.at[c]], bufs.at[b], gsem.at[b])

    def wait_gather(c):
        b = lax.rem(c, NBUF)
        pltpu.make_async_copy(tok_hbm.at[idx_v.at[c]], bufs.at[b], gsem.at[b]).wait()

    def start_scatter(c):
        b = lax.rem(c, NBUF)
        pltpu.async_copy(
            bufs.at[b], out_hbm.at[pl.ds((base + c) * CHUNK, CHUNK)], ssem.at[b])

    def wait_scatter(c):
        b = lax.rem(c, NBUF)
        pltpu.make_async_copy(
            bufs.at[b], out_hbm.at[pl.ds((base + c) * CHUNK, CHUNK)], ssem.at[b]).wait()

    def compute(c):
        b = lax.rem(c, NBUF)
        pos_base = lax.rem(c, PERIOD) * CHUNK

        @plsc.parallel_loop(0, CHUNK, unroll=4)
        def _(r):
            pr = pos_base + r
            for d in range(n_vecs):
                x = pos_v[pr, pl.ds(d * LANES, LANES)]
                plsc.addupdate(bufs.at[b, r, pl.ds(d * LANES, LANES)], x)

    for i in range(LOOK):
        start_gather(i)
    for i in range(LOOK):
        start_gather(i + LOOK)
        wait_gather(i)
        compute(i)
        start_scatter(i)

    def steady(i, carry):
        wait_scatter(i - LOOK)
        start_gather(i + LOOK)
        wait_gather(i)
        compute(i)
        start_scatter(i)
        return carry

    lax.fori_loop(LOOK, n_chunks - LOOK, steady, 0)

    for i in range(n_chunks - LOOK, n_chunks):
        wait_scatter(i - LOOK)
        wait_gather(i)
        compute(i)
        start_scatter(i)
    for i in range(n_chunks - LOOK, n_chunks):
        wait_scatter(i)


def kernel(inputs, token_table, pos_table):
    batch, seq_len = inputs.shape
    d_model = token_table.shape[1]
    total = batch * seq_len
    idx2d = inputs.reshape(total // CHUNK, CHUNK).astype(jnp.int32)
    n_chunks = (total // CHUNK) // NW

    mesh = plsc.VectorSubcoreMesh(core_axis_name="c", subcore_axis_name="s")
    out = pl.kernel(
        _sc_body,
        out_type=jax.ShapeDtypeStruct((total, d_model), jnp.float32),
        mesh=mesh,
        scratch_types=[
            pltpu.VMEM((n_chunks, CHUNK), jnp.int32),
            pltpu.VMEM((seq_len, d_model), jnp.float32),
            pltpu.VMEM((NBUF, CHUNK, d_model), jnp.float32),
            pltpu.SemaphoreType.DMA((NBUF,)),
            pltpu.SemaphoreType.DMA((NBUF,)),
        ],
    )(idx2d, token_table, pos_table)
    return out.reshape(batch, seq_len, d_model)
